# baseline (speedup 1.0000x reference)
import functools

import jax
import jax.numpy as jnp
from jax import lax
from jax.experimental import pallas as pl
from jax.experimental.pallas import tpu as pltpu
from jax.experimental.pallas import tpu_sc as plsc

_NC = 2
_NS = 16
_NW = _NC * _NS
_CHUNK = 32
_SLOTS = 4


def _make_sc_gather_sum(n_rows, dw, n_pad, rpw0, rpw1):
    rpw_max = max(rpw0, rpw1)
    mesh = plsc.VectorSubcoreMesh(
        core_axis_name="c", subcore_axis_name="s",
        num_cores=_NC, num_subcores=_NS)

    @functools.partial(
        pl.kernel,
        out_type=jax.ShapeDtypeStruct((n_pad, 2 * dw), jnp.float32),
        mesh=mesh,
        compiler_params=pltpu.CompilerParams(needs_layout_passes=False),
        scratch_types=[
            [pltpu.VMEM((rpw_max,), jnp.int32) for _ in range(3)],
            [[pltpu.VMEM((_CHUNK, dw), jnp.int32) for _ in range(3)]
             for _ in range(_SLOTS)],
            [pltpu.VMEM((_CHUNK, 2 * dw), jnp.float32)
             for _ in range(_SLOTS)],
            [pltpu.SemaphoreType.DMA for _ in range(_SLOTS)],
            [pltpu.SemaphoreType.DMA for _ in range(_SLOTS)],
        ],
    )
    def sc_gather_sum(table_hbm, nb0_hbm, nb1_hbm, nb2_hbm, gsum_hbm,
                      idx_all, bufs, stage, sems, st_sems):
        cid = lax.axis_index("c")
        sid = lax.axis_index("s")
        rpw = lax.select(cid == 0, rpw0, rpw1)
        base = lax.select(cid == 0, sid * rpw0, _NS * rpw0 + sid * rpw1)
        n_chunks = rpw // _CHUNK
        nbs = (nb0_hbm, nb1_hbm, nb2_hbm)
        for k in range(3):
            pltpu.sync_copy(nbs[k].at[pl.ds(base, rpw_max)], idx_all[k])

        def drain_store(slot):
            pltpu.make_async_copy(
                stage[slot], gsum_hbm.at[pl.ds(0, _CHUNK)],
                st_sems[slot]).wait()

        def fire(ci, slot):
            for k in range(3):
                idx = idx_all[k].at[pl.ds(ci * _CHUNK, _CHUNK)]
                pltpu.async_copy(table_hbm.at[idx], bufs[slot][k], sems[slot])

        def drain(slot):
            for k in range(3):
                pltpu.make_async_copy(
                    table_hbm.at[pl.ds(0, _CHUNK)], bufs[slot][k],
                    sems[slot]).wait()

        hi_mask = jnp.int32(-65536)

        def process(ci, slot):
            b0, b1, b2 = bufs[slot]
            st = stage[slot]

            @pl.when(ci >= _SLOTS)
            def _():
                drain_store(slot)

            def row_body(r, rc):
                for c in range(dw // 16):
                    sl = pl.ds(c * 16, 16)
                    w0 = b0[r, sl]
                    w1 = b1[r, sl]
                    w2 = b2[r, sl]
                    lo = (plsc.bitcast(w0 << 16, jnp.float32)
                          + plsc.bitcast(w1 << 16, jnp.float32)
                          + plsc.bitcast(w2 << 16, jnp.float32))
                    hi = (plsc.bitcast(w0 & hi_mask, jnp.float32)
                          + plsc.bitcast(w1 & hi_mask, jnp.float32)
                          + plsc.bitcast(w2 & hi_mask, jnp.float32))
                    st[r, pl.ds(c * 16, 16)] = lo
                    st[r, pl.ds(dw + c * 16, 16)] = hi
                return rc

            lax.fori_loop(0, _CHUNK, row_body, 0)
            pltpu.async_copy(
                st, gsum_hbm.at[pl.ds(base + ci * _CHUNK, _CHUNK)],
                st_sems[slot])

        for s in range(_SLOTS):
            fire(s, s)

        def group_body(p, carry):
            for s in range(_SLOTS):
                ci = _SLOTS * p + s
                drain(s)
                process(ci, s)

                @pl.when(ci + _SLOTS < n_chunks)
                def _():
                    fire(ci + _SLOTS, s)

            return carry

        lax.fori_loop(0, n_chunks // _SLOTS, group_body, 0)
        for s in range(_SLOTS):
            drain_store(s)

    return sc_gather_sum


def _tc3_body(o1, o2, wc1, wc2, bc, out3):
    out3[...] = (
        jnp.dot(o1[...], wc1[...], preferred_element_type=jnp.float32)
        + jnp.dot(o2[...], wc2[...], preferred_element_type=jnp.float32)
        + bc[...]
    )


def _unpack_halves(wi):
    lo = jax.lax.bitcast_convert_type(wi << 16, jnp.float32)
    hi = jax.lax.bitcast_convert_type(
        wi & jnp.int32(-65536), jnp.float32)
    return lo, hi


def _tc4_body(o2i, g, wae, wao, ba, out4):
    dw = o2i.shape[1]
    o2e, o2o = _unpack_halves(o2i[...])
    gg = g[...]
    fe = ((o2e + gg[:, :dw]) * 0.25).astype(jnp.bfloat16)
    fo = ((o2o + gg[:, dw:]) * 0.25).astype(jnp.bfloat16)
    out4[...] = (
        jnp.dot(fe, wae[...], preferred_element_type=jnp.float32)
        + jnp.dot(fo, wao[...], preferred_element_type=jnp.float32)
        + ba[...]
    )


def kernel(out1, out2, neighbour, W_comb, b_comb, W_agg, b_agg):
    n, d = out2.shape
    dw = d // 2
    dout = b_comb.shape[0]

    out1_bf = out1.astype(jnp.bfloat16)
    out2_bf = out2.astype(jnp.bfloat16)
    out2_i = jax.lax.bitcast_convert_type(
        out2_bf.reshape(n, dw, 2), jnp.int32)

    step = _SLOTS * _CHUNK
    per_pair = ((n + _NS - 1) // _NS + step - 1) // step * step
    rpw1 = max(step, int(round(per_pair * 0.265 / step)) * step)
    rpw0 = per_pair - rpw1
    n_pad = _NS * per_pair
    nbt = jnp.transpose(neighbour.astype(jnp.int32))
    nbt = jnp.pad(nbt, ((0, 0), (0, n_pad + max(rpw0, rpw1) - n)))
    gsum = _make_sc_gather_sum(n, dw, n_pad, rpw0, rpw1)(
        out2_i, nbt[0], nbt[1], nbt[2])

    wcT = jnp.transpose(W_comb[:, :, 0]).astype(jnp.bfloat16)
    wc1 = wcT[:d]
    wc2 = wcT[d:]
    wa = jnp.transpose(W_agg[:, :, 0]).astype(jnp.bfloat16)
    wae = wa[0::2]
    wao = wa[1::2]
    bc = b_comb.reshape(1, dout)
    ba = b_agg.reshape(1, dout)

    blk = 2000
    assert n % blk == 0
    grid = (n // blk,)
    row_spec = pl.BlockSpec((blk, d), lambda i: (i, 0))
    gi_spec = pl.BlockSpec((blk, dw), lambda i: (i, 0))
    out_spec = pl.BlockSpec((blk, dout), lambda i: (i, 0))
    full = lambda s: pl.BlockSpec(s, lambda i: (0, 0))
    out_ty = jax.ShapeDtypeStruct((n, dout), jnp.float32)
    out3 = pl.pallas_call(
        _tc3_body,
        grid=grid,
        in_specs=[row_spec, row_spec, full((d, dout)), full((d, dout)),
                  full((1, dout))],
        out_specs=out_spec,
        out_shape=out_ty,
    )(out1_bf, out2_bf, wc1, wc2, bc)
    out4 = pl.pallas_call(
        _tc4_body,
        grid=grid,
        in_specs=[gi_spec, row_spec, full((dw, dout)), full((dw, dout)),
                  full((1, dout))],
        out_specs=out_spec,
        out_shape=out_ty,
    )(out2_i, gsum, wae, wao, ba)
    return (out3, out4)


# speedup vs baseline: 2.0865x; 2.0865x over previous
import functools

import jax
import jax.numpy as jnp
from jax import lax
from jax.experimental import pallas as pl
from jax.experimental.pallas import tpu as pltpu
from jax.experimental.pallas import tpu_sc as plsc

_NC = 2
_NS = 16
_NW = _NC * _NS
_CHUNK = 32
_SLOTS = 4
_CORE1_FRAC = 0.265


def _make_sc_gather_sum(n_rows, d, n_pad, rpw0, rpw1):
    rpw_max = max(rpw0, rpw1)
    mesh = plsc.VectorSubcoreMesh(
        core_axis_name="c", subcore_axis_name="s",
        num_cores=_NC, num_subcores=_NS)

    @functools.partial(
        pl.kernel,
        out_type=jax.ShapeDtypeStruct((n_pad, d), jnp.float32),
        mesh=mesh,
        scratch_types=[
            [pltpu.VMEM((rpw_max,), jnp.int32) for _ in range(3)],
            [[pltpu.VMEM((_CHUNK, d), jnp.float32) for _ in range(3)]
             for _ in range(_SLOTS)],
            [pltpu.SemaphoreType.DMA for _ in range(_SLOTS)],
            [pltpu.SemaphoreType.DMA for _ in range(_SLOTS)],
        ],
    )
    def sc_gather_sum(out2_hbm, nb0_hbm, nb1_hbm, nb2_hbm, gsum_hbm,
                      idx_all, bufs, sems, st_sems):
        cid = lax.axis_index("c")
        sid = lax.axis_index("s")
        rpw = lax.select(cid == 0, rpw0, rpw1)
        base = lax.select(cid == 0, sid * rpw0, _NS * rpw0 + sid * rpw1)
        n_chunks = rpw // _CHUNK
        nbs = (nb0_hbm, nb1_hbm, nb2_hbm)
        for k in range(3):
            pltpu.sync_copy(nbs[k].at[pl.ds(base, rpw_max)], idx_all[k])

        def drain_store(slot):
            pltpu.make_async_copy(
                bufs[slot][0], gsum_hbm.at[pl.ds(0, _CHUNK)],
                st_sems[slot]).wait()

        def fire(ci, slot, first=False):
            for k in (1, 2):
                idx = idx_all[k].at[pl.ds(ci * _CHUNK, _CHUNK)]
                pltpu.async_copy(out2_hbm.at[idx], bufs[slot][k], sems[slot])
            if not first:
                drain_store(slot)
            idx = idx_all[0].at[pl.ds(ci * _CHUNK, _CHUNK)]
            pltpu.async_copy(out2_hbm.at[idx], bufs[slot][0], sems[slot])

        def drain(slot):
            for k in range(3):
                pltpu.make_async_copy(
                    out2_hbm.at[pl.ds(0, _CHUNK)], bufs[slot][k],
                    sems[slot]).wait()

        def process(ci, slot):
            b0, b1, b2 = bufs[slot]

            def row_body(r, rc):
                for c in range(d // 16):
                    sl = pl.ds(c * 16, 16)
                    plsc.addupdate(b0.at[r, sl], b1[r, sl] + b2[r, sl])
                return rc

            lax.fori_loop(0, _CHUNK, row_body, 0)
            pltpu.async_copy(
                b0, gsum_hbm.at[pl.ds(base + ci * _CHUNK, _CHUNK)],
                st_sems[slot])

        for s in range(_SLOTS):
            fire(s, s, first=True)

        def group_body(p, carry):
            for s in range(_SLOTS):
                ci = _SLOTS * p + s
                drain(s)
                process(ci, s)

                @pl.when(ci + _SLOTS < n_chunks)
                def _():
                    fire(ci + _SLOTS, s)

            return carry

        lax.fori_loop(0, n_chunks // _SLOTS, group_body, 0)
        for s in range(_SLOTS):
            drain_store(s)

    return sc_gather_sum


def _tc3_body(o1, o2, wc1, wc2, bc, out3):
    out3[...] = (
        jnp.dot(o1[...].astype(jnp.bfloat16), wc1[...],
                preferred_element_type=jnp.float32)
        + jnp.dot(o2[...].astype(jnp.bfloat16), wc2[...],
                  preferred_element_type=jnp.float32)
        + bc[...]
    )


def _tc4_body(o2, g, wa, ba, out4):
    f = ((o2[...] + g[...]) * 0.25).astype(jnp.bfloat16)
    out4[...] = jnp.dot(f, wa[...], preferred_element_type=jnp.float32) + ba[...]


def kernel(out1, out2, neighbour, W_comb, b_comb, W_agg, b_agg):
    n, d = out2.shape
    dout = b_comb.shape[0]

    step = _SLOTS * _CHUNK
    per_pair = ((n + _NS - 1) // _NS + step - 1) // step * step
    rpw1 = max(step, int(round(per_pair * _CORE1_FRAC / step)) * step)
    rpw0 = per_pair - rpw1
    n_pad = _NS * per_pair
    nbt = jnp.transpose(neighbour.astype(jnp.int32))
    nbt = jnp.pad(nbt, ((0, 0), (0, n_pad + max(rpw0, rpw1) - n)))
    gsum = _make_sc_gather_sum(n, d, n_pad, rpw0, rpw1)(
        out2, nbt[0], nbt[1], nbt[2])

    wcT = jnp.transpose(W_comb[:, :, 0]).astype(jnp.bfloat16)
    wc1 = wcT[:d]
    wc2 = wcT[d:]
    wa = jnp.transpose(W_agg[:, :, 0]).astype(jnp.bfloat16)
    bc = b_comb.reshape(1, dout)
    ba = b_agg.reshape(1, dout)

    blk = 2000
    assert n % blk == 0
    grid = (n // blk,)
    row_spec = pl.BlockSpec((blk, d), lambda i: (i, 0))
    out_spec = pl.BlockSpec((blk, dout), lambda i: (i, 0))
    full = lambda s: pl.BlockSpec(s, lambda i: (0, 0))
    out_ty = jax.ShapeDtypeStruct((n, dout), jnp.float32)
    out3 = pl.pallas_call(
        _tc3_body,
        grid=grid,
        in_specs=[row_spec, row_spec, full((d, dout)), full((d, dout)),
                  full((1, dout))],
        out_specs=out_spec,
        out_shape=out_ty,
    )(out1, out2, wc1, wc2, bc)
    out4 = pl.pallas_call(
        _tc4_body,
        grid=grid,
        in_specs=[row_spec, row_spec, full((d, dout)), full((1, dout))],
        out_specs=out_spec,
        out_shape=out_ty,
    )(out2, gsum, wa, ba)
    return (out3, out4)
